# baseline (speedup 1.0000x reference)
import functools

import jax
import jax.numpy as jnp
from jax import lax
from jax.experimental import pallas as pl
from jax.experimental.pallas import tpu as pltpu
from jax.experimental.pallas import tpu_sc as plsc

_NC = 2
_NS = 16
_NW = _NC * _NS

_CHUNK = 8192
_L = 16


def _lookup_call(B, H, D, Vp):
    n = (B * H) // _CHUNK
    cols = B // _CHUNK
    mesh = plsc.VectorSubcoreMesh(core_axis_name="c", subcore_axis_name="s")

    @functools.partial(
        pl.kernel,
        mesh=mesh,
        out_type=jax.ShapeDtypeStruct((H, D, B), jnp.int32),
        scratch_types=[
            pltpu.VMEM((Vp,), jnp.int32),
            pltpu.VMEM((_CHUNK,), jnp.int32),
            pltpu.VMEM((_CHUNK,), jnp.int32),
            pltpu.VMEM((_CHUNK,), jnp.int32),
            pltpu.SemaphoreType.DMA,
            pltpu.SemaphoreType.DMA,
            pltpu.SemaphoreType.DMA,
            pltpu.SemaphoreType.DMA,
            pltpu.SemaphoreType.DMA,
            pltpu.SemaphoreType.DMA,
        ],
        compiler_params=pltpu.CompilerParams(
            use_tc_tiling_on_sc=False, needs_layout_passes=False),
    )
    def run(tab_hbm, idx_hbm, out_hbm, plane, buf0, buf1, buf2,
            isem0, isem1, isem2, ssem0, ssem1, ssem2):
        wid = lax.axis_index("s") * _NC + lax.axis_index("c")

        pltpu.sync_copy(tab_hbm.at[wid], plane)

        bufs = (buf0, buf1, buf2)
        isems = (isem0, isem1, isem2)
        ssems = (ssem0, ssem1, ssem2)

        def idx_cp(g, b):
            h = g // cols
            c = g % cols
            return pltpu.make_async_copy(
                idx_hbm.at[h, pl.ds(c * _CHUNK, _CHUNK)], bufs[b], isems[b])

        def store_cp(g, b):
            h = g // cols
            c = g % cols
            return pltpu.make_async_copy(
                bufs[b], out_hbm.at[h, wid, pl.ds(c * _CHUNK, _CHUNK)],
                ssems[b])

        def gather_chunk(buf):
            w = 4
            for k in range(_CHUNK // (_L * w)):
                ivs = [buf[pl.ds((k * w + j) * _L, _L)] for j in range(w)]
                rvs = [plsc.load_gather(plane, [iv]) for iv in ivs]
                for j in range(w):
                    buf[pl.ds((k * w + j) * _L, _L)] = rvs[j]

        idx_cp(0, 0).start()
        idx_cp(1, 1).start()

        def step(g, b):
            idx_cp(g, b).wait()
            gather_chunk(bufs[b])
            store_cp(g, b).start()

            s = (b + 2) % 3

            @pl.when(g + 2 < n)
            def _():
                @pl.when(g >= 1)
                def _():
                    store_cp(g - 1, s).wait()
                idx_cp(g + 2, s).start()

        def body(t, carry):
            step(3 * t, 0)
            step(3 * t + 1, 1)
            step(3 * t + 2, 2)
            return carry

        lax.fori_loop(0, n // 3, body, 0)
        step(n - 1, (n - 1) % 3)

        store_cp(n - 3, (n - 3) % 3).wait()
        store_cp(n - 2, (n - 2) % 3).wait()
        store_cp(n - 1, (n - 1) % 3).wait()

    return run


def kernel(X, table):
    B, H = X.shape
    V, D = table.shape
    Vp = ((V + 127) // 128) * 128
    idx = X.T
    tab = jnp.pad(
        lax.bitcast_convert_type(table, jnp.int32).T,
        ((0, 0), (0, Vp - V)))
    out_t = _lookup_call(B, H, D, Vp)(tab, idx)
    return lax.bitcast_convert_type(out_t.transpose(2, 0, 1), jnp.float32)
